# baseline (speedup 1.0000x reference)
import jax
import jax.numpy as jnp
from jax.experimental import pallas as pl
from jax.experimental.pallas import tpu as pltpu

B = 1024
D = 256
H = 256
K = 16384
BK = 512
NSTEPS = K // BK
NA = 18
NA_PAD = 32


def _fused(q_ref, k_ref, k2e_ref, v_ref, h0_ref, c0_ref, wih_ref, whh_ref, b1_ref, b2_ref,
           wa_ref, ba_ref, wc_ref, bc_ref,
           act_ref, val_ref, h_ref, c_ref,
           acc_ref, m_ref, d_ref):
    j = pl.program_id(0)

    @pl.when(j == 0)
    def _init():
        m_ref[...] = jnp.full_like(m_ref, -3.0e38)
        d_ref[...] = jnp.zeros_like(d_ref)
        acc_ref[...] = jnp.zeros_like(acc_ref)

    q = q_ref[...]
    k = k_ref[...]
    v = v_ref[...]

    s = jax.lax.dot_general(q, k, (((1,), (1,)), ((), ())),
                            preferred_element_type=jnp.float32
                            ) * 2.8853900817779268 - k2e_ref[...]

    m_old = m_ref[...]
    m_new = jnp.maximum(m_old, jnp.max(s, axis=1, keepdims=True))
    alpha = jnp.exp2(m_old - m_new)
    p = jnp.exp2(s - m_new)
    d_new = d_ref[...] * alpha + jnp.sum(p, axis=1, keepdims=True)
    acc_new = acc_ref[...] * alpha + jax.lax.dot_general(
        p, v, (((1,), (0,)), ((), ())),
        preferred_element_type=jnp.float32)
    m_ref[...] = m_new
    d_ref[...] = d_new
    acc_ref[...] = acc_new

    @pl.when(j == NSTEPS - 1)
    def _epilogue():
        m_t = acc_new / d_new

        h0 = h0_ref[...]
        c0 = c0_ref[...]
        gates = (jax.lax.dot_general(q, wih_ref[...], (((1,), (1,)), ((), ())),
                                     preferred_element_type=jnp.float32)
                 + jax.lax.dot_general(h0, whh_ref[...], (((1,), (1,)), ((), ())),
                                       preferred_element_type=jnp.float32)
                 + b1_ref[...] + b2_ref[...])
        i_g = jax.nn.sigmoid(gates[:, 0:H])
        f_g = jax.nn.sigmoid(gates[:, H:2 * H])
        g_g = jnp.tanh(gates[:, 2 * H:3 * H])
        o_g = jax.nn.sigmoid(gates[:, 3 * H:4 * H])
        c_t = f_g * c0 + i_g * g_g + m_t
        h_t = o_g * jnp.tanh(c_t)
        c_ref[...] = c_t
        h_ref[...] = h_t

        logits = jax.lax.dot_general(h_t, wa_ref[...], (((1,), (1,)), ((), ())),
                                     preferred_element_type=jnp.float32) + ba_ref[...]
        lmax = jnp.max(logits, axis=1, keepdims=True)
        e = jnp.exp(logits - lmax)
        act_ref[...] = e / jnp.sum(e, axis=1, keepdims=True)

        val_ref[...] = (jnp.sum(h_t * wc_ref[...], axis=1, keepdims=True)
                        + bc_ref[...])


def kernel(x_t, h0, c0, keys_mem, vals_mem, W_ih, W_hh, b_ih, b_hh,
           W_actor, b_actor, W_critic, b_critic):
    b, s_len, d = x_t.shape
    q = x_t.reshape(b, d)
    k2e = (jnp.sum(keys_mem * keys_mem, axis=1) * 1.4426950408889634).reshape(1, K)
    b1 = b_ih.reshape(1, 4 * H)
    b2 = b_hh.reshape(1, 4 * H)
    ba = b_actor.reshape(1, NA)
    wc = W_critic.reshape(1, H)
    bc = b_critic.reshape(1, 1)

    full = lambda shp: pl.BlockSpec(shp, lambda j: (0, 0))
    out = pl.pallas_call(
        _fused,
        grid=(NSTEPS,),
        in_specs=[
            full((B, D)),
            pl.BlockSpec((BK, D), lambda j: (j, 0)),
            pl.BlockSpec((1, BK), lambda j: (0, j)),
            pl.BlockSpec((BK, H), lambda j: (j, 0)),
            full((B, H)),
            full((B, H)),
            full((4 * H, D)),
            full((4 * H, H)),
            full((1, 4 * H)),
            full((1, 4 * H)),
            full((NA, H)),
            full((1, NA)),
            full((1, H)),
            full((1, 1)),
        ],
        out_specs=[
            full((B, NA)),
            full((B, 1)),
            full((B, H)),
            full((B, H)),
        ],
        out_shape=[
            jax.ShapeDtypeStruct((B, NA), jnp.float32),
            jax.ShapeDtypeStruct((B, 1), jnp.float32),
            jax.ShapeDtypeStruct((B, H), jnp.float32),
            jax.ShapeDtypeStruct((B, H), jnp.float32),
        ],
        scratch_shapes=[
            pltpu.VMEM((B, H), jnp.float32),
            pltpu.VMEM((B, 1), jnp.float32),
            pltpu.VMEM((B, 1), jnp.float32),
        ],
        compiler_params=pltpu.CompilerParams(
            dimension_semantics=("arbitrary",),
        ),
    )(q, keys_mem, k2e, vals_mem, h0[0], c0[0], W_ih, W_hh, b1, b2, W_actor, ba, wc, bc)

    act, val, h_t, c_t = out
    action_dist = act.reshape(b, s_len, NA)
    value = val.reshape(b, s_len, 1)
    h_seq = h_t.reshape(b, s_len, H)
    c_out = c_t.reshape(1, b, H)
    return (action_dist, value, h_seq, c_out)


# speedup vs baseline: 1.6761x; 1.6761x over previous
import jax
import jax.numpy as jnp
from jax.experimental import pallas as pl
from jax.experimental.pallas import tpu as pltpu

B = 1024
D = 256
H = 256
K = 16384
BK = 1024
NSTEPS = K // BK
NA = 18
NA_PAD = 32


def _fused(q_ref, k_ref, v_ref, h0_ref, c0_ref, wih_ref, whh_ref, b1_ref, b2_ref,
           wa_ref, ba_ref, wc_ref, bc_ref,
           act_ref, val_ref, h_ref, c_ref,
           acc_ref, m_ref, d_ref):
    j = pl.program_id(0)

    @pl.when(j == 0)
    def _init():
        m_ref[...] = jnp.full_like(m_ref, -3.0e38)
        d_ref[...] = jnp.zeros_like(d_ref)
        acc_ref[...] = jnp.zeros_like(acc_ref)

    q = q_ref[...]
    k = k_ref[...]
    v = v_ref[...]

    k2e = (jnp.sum(k * k, axis=1, keepdims=True)
           * 1.4426950408889634).reshape(1, BK)
    s = jax.lax.dot_general(q, k, (((1,), (1,)), ((), ())),
                            preferred_element_type=jnp.float32
                            ) * 2.8853900817779268 - k2e

    m_old = m_ref[...]
    m_new = jnp.maximum(m_old, jnp.max(s, axis=1, keepdims=True))
    alpha = jnp.exp2(m_old - m_new)
    p = jnp.exp2(s - m_new)
    d_new = d_ref[...] * alpha + jnp.sum(p, axis=1, keepdims=True)
    acc_new = acc_ref[...] * alpha + jax.lax.dot_general(
        p, v, (((1,), (0,)), ((), ())),
        preferred_element_type=jnp.float32)
    m_ref[...] = m_new
    d_ref[...] = d_new
    acc_ref[...] = acc_new

    @pl.when(j == NSTEPS - 1)
    def _epilogue():
        m_t = acc_new / d_new

        h0 = h0_ref[...]
        c0 = c0_ref[...]
        gates = (jax.lax.dot_general(q, wih_ref[...], (((1,), (1,)), ((), ())),
                                     preferred_element_type=jnp.float32)
                 + jax.lax.dot_general(h0, whh_ref[...], (((1,), (1,)), ((), ())),
                                       preferred_element_type=jnp.float32)
                 + b1_ref[...] + b2_ref[...])
        i_g = jax.nn.sigmoid(gates[:, 0:H])
        f_g = jax.nn.sigmoid(gates[:, H:2 * H])
        g_g = jnp.tanh(gates[:, 2 * H:3 * H])
        o_g = jax.nn.sigmoid(gates[:, 3 * H:4 * H])
        c_t = f_g * c0 + i_g * g_g + m_t
        h_t = o_g * jnp.tanh(c_t)
        c_ref[...] = c_t
        h_ref[...] = h_t

        logits = jax.lax.dot_general(h_t, wa_ref[...], (((1,), (1,)), ((), ())),
                                     preferred_element_type=jnp.float32) + ba_ref[...]
        lmax = jnp.max(logits, axis=1, keepdims=True)
        e = jnp.exp(logits - lmax)
        act_ref[...] = e / jnp.sum(e, axis=1, keepdims=True)

        val_ref[...] = (jnp.sum(h_t * wc_ref[...], axis=1, keepdims=True)
                        + bc_ref[...])


def kernel(x_t, h0, c0, keys_mem, vals_mem, W_ih, W_hh, b_ih, b_hh,
           W_actor, b_actor, W_critic, b_critic):
    b, s_len, d = x_t.shape
    q = x_t.reshape(b, d)
    b1 = b_ih.reshape(1, 4 * H)
    b2 = b_hh.reshape(1, 4 * H)
    ba = b_actor.reshape(1, NA)
    wc = W_critic.reshape(1, H)
    bc = b_critic.reshape(1, 1)

    full = lambda shp: pl.BlockSpec(shp, lambda j: (0, 0))
    out = pl.pallas_call(
        _fused,
        grid=(NSTEPS,),
        in_specs=[
            full((B, D)),
            pl.BlockSpec((BK, D), lambda j: (j, 0)),
            pl.BlockSpec((BK, H), lambda j: (j, 0)),
            full((B, H)),
            full((B, H)),
            full((4 * H, D)),
            full((4 * H, H)),
            full((1, 4 * H)),
            full((1, 4 * H)),
            full((NA, H)),
            full((1, NA)),
            full((1, H)),
            full((1, 1)),
        ],
        out_specs=[
            full((B, NA)),
            full((B, 1)),
            full((B, H)),
            full((B, H)),
        ],
        out_shape=[
            jax.ShapeDtypeStruct((B, NA), jnp.float32),
            jax.ShapeDtypeStruct((B, 1), jnp.float32),
            jax.ShapeDtypeStruct((B, H), jnp.float32),
            jax.ShapeDtypeStruct((B, H), jnp.float32),
        ],
        scratch_shapes=[
            pltpu.VMEM((B, H), jnp.float32),
            pltpu.VMEM((B, 1), jnp.float32),
            pltpu.VMEM((B, 1), jnp.float32),
        ],
        compiler_params=pltpu.CompilerParams(
            dimension_semantics=("arbitrary",),
        ),
    )(q, keys_mem, vals_mem, h0[0], c0[0], W_ih, W_hh, b1, b2, W_actor, ba, wc, bc)

    act, val, h_t, c_t = out
    action_dist = act.reshape(b, s_len, NA)
    value = val.reshape(b, s_len, 1)
    h_seq = h_t.reshape(b, s_len, H)
    c_out = c_t.reshape(1, b, H)
    return (action_dist, value, h_seq, c_out)
